# baseline (speedup 1.0000x reference)
import functools

import jax
import jax.numpy as jnp
from jax import lax
from jax.experimental import pallas as pl
from jax.experimental.pallas import tpu as pltpu
from jax.experimental.pallas import tpu_sc as plsc

N = 10000
E = 320000
D = 128
R = 8
B = 4

NC = 2
NS = 16
NW = NC * NS

EDGES_PER_TILE = E // NW
CHUNK = 100
NCHUNK = EDGES_PER_TILE // CHUNK
ROW_BLK = 48
ROW_REM = 16
ROWS_PER_TILE = 624


def _tc_prep_body(comps_ref, x_ref, bases_ref, out_ref):
    r = pl.program_id(0)
    w = jnp.zeros((D, D), dtype=jnp.float32)
    for b in range(B):
        w = w + comps_ref[r, b] * bases_ref[b]
    out_ref[...] = jnp.dot(x_ref[...], w, preferred_element_type=jnp.float32)


def _tc_prep(X, bases, comps):
    grid = (R,)
    return pl.pallas_call(
        _tc_prep_body,
        grid=grid,
        in_specs=[
            pl.BlockSpec(memory_space=pltpu.SMEM),
            pl.BlockSpec((N, D), lambda r: (0, 0)),
            pl.BlockSpec((B, D, D), lambda r: (0, 0, 0)),
        ],
        out_specs=pl.BlockSpec((N, D), lambda r: (r, 0)),
        out_shape=jax.ShapeDtypeStruct((R * N, D), jnp.float32),
    )(comps, X, bases)


def _tc_idxc_body(src_ref, typ_ref, dst_ref, out_ref):
    out_ref[:, 0, :] = typ_ref[:, 0, :] * N + src_ref[:, 0, :]
    out_ref[:, 1, :] = dst_ref[:, 0, :]


def _tc_idxc(src, typ, dst):
    nchunks = NW * NCHUNK
    src3 = src.reshape(nchunks, 1, CHUNK)
    typ3 = typ.reshape(nchunks, 1, CHUNK)
    dst3 = dst.reshape(nchunks, 1, CHUNK)
    return pl.pallas_call(
        _tc_idxc_body,
        out_shape=jax.ShapeDtypeStruct((nchunks, 2, CHUNK), jnp.int32),
    )(src3, typ3, dst3)


def _sc_edges_body(xw_hbm, idxc_hbm, acc_hbm, deg_hbm,
                   idxp_v, rowsA_v, rowsB_v, zrow_v, deg_loc, acc_sh,
                   gsemA, gsemB, ssemA, ssemB):
    c = lax.axis_index("c")
    s = lax.axis_index("s")
    wid = c * NS + s

    zero16 = jnp.zeros((16,), jnp.float32)
    ones16 = jnp.ones((16,), jnp.float32)

    def init_zrow(i, _):
        zrow_v[i // (D // 16), pl.ds((i % (D // 16)) * 16, 16)] = zero16
        return 0
    lax.fori_loop(0, ROW_BLK * (D // 16), init_zrow, 0)

    def init_deg(i, _):
        deg_loc[pl.ds(i * 16, 16)] = zero16
        return 0
    lax.fori_loop(0, N // 16, init_deg, 0)

    row0 = s * ROWS_PER_TILE

    def zero_body(i, _):
        pltpu.sync_copy(zrow_v, acc_sh.at[pl.ds(row0 + i * ROW_BLK, ROW_BLK)])
        return 0
    lax.fori_loop(0, ROWS_PER_TILE // ROW_BLK, zero_body, 0)

    @pl.when(s == NS - 1)
    def _():
        pltpu.sync_copy(zrow_v.at[pl.ds(0, ROW_REM)],
                        acc_sh.at[pl.ds(NS * ROWS_PER_TILE, ROW_REM)])
    plsc.subcore_barrier()

    def count_deg(q, j):
        for i in range(CHUNK // 16):
            idx16 = idxp_v[q, j, 1, pl.ds(i * 16, 16)]
            plsc.addupdate_scatter(deg_loc, [idx16], ones16)
        rem = CHUNK % 16
        if rem:
            lanes = lax.broadcasted_iota(jnp.int32, (16,), 0)
            idx16 = idxp_v[q, j, 1, pl.ds(CHUNK - 16, 16)]
            plsc.addupdate_scatter(deg_loc, [idx16], ones16,
                                   mask=lanes >= (16 - rem))

    def gather(q, j, rows, sem):
        pltpu.async_copy(xw_hbm.at[idxp_v.at[q, j, 0]], rows, sem)

    def wait_gather(q, j, rows, sem):
        pltpu.make_async_copy(xw_hbm.at[idxp_v.at[q, j, 0]], rows, sem).wait()

    def scatter(q, j, rows, sem):
        pltpu.async_copy(rows, acc_sh.at[idxp_v.at[q, j, 1]], sem, add=True)
        count_deg(q, j)

    def wait_scatter(q, j, rows, sem):
        pltpu.make_async_copy(rows, acc_sh.at[idxp_v.at[q, j, 1]], sem).wait()

    pltpu.sync_copy(idxc_hbm.at[wid, pl.ds(0, 2)], idxp_v.at[0])
    gather(0, 0, rowsA_v, gsemA)

    def quad_body(i, _):
        g = i * 4
        pltpu.sync_copy(idxc_hbm.at[wid, pl.ds(g + 2, 2)], idxp_v.at[1])
        gather(0, 1, rowsB_v, gsemB)
        wait_gather(0, 0, rowsA_v, gsemA)
        scatter(0, 0, rowsA_v, ssemA)
        wait_gather(0, 1, rowsB_v, gsemB)
        scatter(0, 1, rowsB_v, ssemB)
        wait_scatter(0, 0, rowsA_v, ssemA)
        gather(1, 0, rowsA_v, gsemA)
        wait_scatter(0, 1, rowsB_v, ssemB)
        gather(1, 1, rowsB_v, gsemB)

        @pl.when(g + 4 < NCHUNK)
        def _():
            pltpu.sync_copy(idxc_hbm.at[wid, pl.ds(g + 4, 2)], idxp_v.at[0])
        wait_gather(1, 0, rowsA_v, gsemA)
        scatter(1, 0, rowsA_v, ssemA)
        wait_gather(1, 1, rowsB_v, gsemB)
        scatter(1, 1, rowsB_v, ssemB)
        wait_scatter(1, 0, rowsA_v, ssemA)

        @pl.when(g + 4 < NCHUNK)
        def _():
            gather(0, 0, rowsA_v, gsemA)
        wait_scatter(1, 1, rowsB_v, ssemB)
        return 0

    lax.fori_loop(0, NCHUNK // 4, quad_body, 0)

    pltpu.sync_copy(deg_loc, deg_hbm.at[c, s])
    plsc.subcore_barrier()

    def out_body(i, _):
        sl = pl.ds(row0 + i * ROW_BLK, ROW_BLK)
        pltpu.sync_copy(acc_sh.at[sl], acc_hbm.at[c, sl])
        return 0
    lax.fori_loop(0, ROWS_PER_TILE // ROW_BLK, out_body, 0)

    @pl.when(s == NS - 1)
    def _():
        sl = pl.ds(NS * ROWS_PER_TILE, ROW_REM)
        pltpu.sync_copy(acc_sh.at[sl], acc_hbm.at[c, sl])


@functools.partial(
    pl.kernel,
    out_type=(
        jax.ShapeDtypeStruct((NC, N, D), jnp.float32),
        jax.ShapeDtypeStruct((NC, NS, N), jnp.float32),
    ),
    mesh=plsc.VectorSubcoreMesh(core_axis_name="c", subcore_axis_name="s",
                                num_cores=NC, num_subcores=NS),
    compiler_params=pltpu.CompilerParams(needs_layout_passes=False),
    scratch_types=[
        pltpu.VMEM((2, 2, 2, CHUNK), jnp.int32),
        pltpu.VMEM((CHUNK, D), jnp.float32),
        pltpu.VMEM((CHUNK, D), jnp.float32),
        pltpu.VMEM((ROW_BLK, D), jnp.float32),
        pltpu.VMEM((N,), jnp.float32),
        pltpu.VMEM_SHARED((N, D), jnp.float32),
        pltpu.SemaphoreType.DMA,
        pltpu.SemaphoreType.DMA,
        pltpu.SemaphoreType.DMA,
        pltpu.SemaphoreType.DMA,
    ],
)
def _sc_edges(xw_hbm, idxc_hbm, acc_hbm, deg_hbm,
              idxp_v, rowsA_v, rowsB_v, zrow_v, deg_loc, acc_sh,
              gsemA, gsemB, ssemA, ssemB):
    _sc_edges_body(xw_hbm, idxc_hbm, acc_hbm, deg_hbm,
                   idxp_v, rowsA_v, rowsB_v, zrow_v, deg_loc, acc_sh,
                   gsemA, gsemB, ssemA, ssemB)


def _tc_finish_body(acc_ref, deg_ref, bias_ref, out_ref):
    acc = acc_ref[0] + acc_ref[1]
    deg = jnp.sum(deg_ref[...], axis=0)[:, None]
    norm = 1.0 / jnp.clip(deg, 1.0, None)
    out_ref[...] = jnp.maximum(acc * norm + bias_ref[...], 0.0)


def _tc_finish(acc, deg, bias):
    return pl.pallas_call(
        _tc_finish_body,
        out_shape=jax.ShapeDtypeStruct((N, D), jnp.float32),
    )(acc, deg.reshape(NC * NS, N), bias.reshape(1, D))


def kernel(X, edge_index, edge_type, bases, comps, bias):
    src = edge_index[0]
    dst = edge_index[1]
    xw = _tc_prep(X, bases, comps)
    idxc = _tc_idxc(src, edge_type, dst).reshape(NW, NCHUNK, 2, CHUNK)
    acc, deg = _sc_edges(xw, idxc)
    return _tc_finish(acc, deg, bias)


# speedup vs baseline: 1.0105x; 1.0105x over previous
import functools

import jax
import jax.numpy as jnp
from jax import lax
from jax.experimental import pallas as pl
from jax.experimental.pallas import tpu as pltpu
from jax.experimental.pallas import tpu_sc as plsc

N = 10000
E = 320000
D = 128
R = 8
B = 4

NC = 2
NS = 16
NW = NC * NS

EDGES_PER_TILE = E // NW
CHUNK = 100
NCHUNK = EDGES_PER_TILE // CHUNK
ROW_BLK = 48
ROW_REM = 16
ROWS_PER_TILE = 624


def _tc_prep_body(comps_ref, x_ref, bases_ref, src_ref, typ_ref, dst_ref,
                  xw_ref, idxc_ref):
    r = pl.program_id(0)
    w = jnp.zeros((D, D), dtype=jnp.float32)
    for b in range(B):
        w = w + comps_ref[r, b] * bases_ref[b]
    xw_ref[...] = jnp.dot(x_ref[...], w, preferred_element_type=jnp.float32)

    @pl.when(r == 0)
    def _():
        idxc_ref[:, 0, :] = typ_ref[:, 0, :] * N + src_ref[:, 0, :]
        idxc_ref[:, 1, :] = dst_ref[:, 0, :]


def _tc_prep(X, bases, comps, src, typ, dst):
    nchunks = NW * NCHUNK
    src3 = src.reshape(nchunks, 1, CHUNK)
    typ3 = typ.reshape(nchunks, 1, CHUNK)
    dst3 = dst.reshape(nchunks, 1, CHUNK)
    grid = (R,)
    return pl.pallas_call(
        _tc_prep_body,
        grid=grid,
        in_specs=[
            pl.BlockSpec(memory_space=pltpu.SMEM),
            pl.BlockSpec((N, D), lambda r: (0, 0)),
            pl.BlockSpec((B, D, D), lambda r: (0, 0, 0)),
            pl.BlockSpec((nchunks, 1, CHUNK), lambda r: (0, 0, 0)),
            pl.BlockSpec((nchunks, 1, CHUNK), lambda r: (0, 0, 0)),
            pl.BlockSpec((nchunks, 1, CHUNK), lambda r: (0, 0, 0)),
        ],
        out_specs=[
            pl.BlockSpec((N, D), lambda r: (r, 0)),
            pl.BlockSpec((nchunks, 2, CHUNK), lambda r: (0, 0, 0)),
        ],
        out_shape=[
            jax.ShapeDtypeStruct((R * N, D), jnp.float32),
            jax.ShapeDtypeStruct((nchunks, 2, CHUNK), jnp.int32),
        ],
    )(comps, X, bases, src3, typ3, dst3)


def _sc_edges_body(xw_hbm, idxc_hbm, acc_hbm, deg_hbm,
                   idxp_v, rowsA_v, rowsB_v, zrow_v, deg_loc, acc_sh,
                   gsemA, gsemB, ssemA, ssemB):
    c = lax.axis_index("c")
    s = lax.axis_index("s")
    wid = c * NS + s

    zero16 = jnp.zeros((16,), jnp.float32)
    ones16 = jnp.ones((16,), jnp.float32)

    def init_zrow(i, _):
        zrow_v[i // (D // 16), pl.ds((i % (D // 16)) * 16, 16)] = zero16
        return 0
    lax.fori_loop(0, ROW_BLK * (D // 16), init_zrow, 0)

    def init_deg(i, _):
        deg_loc[pl.ds(i * 16, 16)] = zero16
        return 0
    lax.fori_loop(0, N // 16, init_deg, 0)

    row0 = s * ROWS_PER_TILE

    def zero_body(i, _):
        pltpu.sync_copy(zrow_v, acc_sh.at[pl.ds(row0 + i * ROW_BLK, ROW_BLK)])
        return 0
    lax.fori_loop(0, ROWS_PER_TILE // ROW_BLK, zero_body, 0)

    @pl.when(s == NS - 1)
    def _():
        pltpu.sync_copy(zrow_v.at[pl.ds(0, ROW_REM)],
                        acc_sh.at[pl.ds(NS * ROWS_PER_TILE, ROW_REM)])
    plsc.subcore_barrier()

    def count_deg(q, j):
        for i in range(CHUNK // 16):
            idx16 = idxp_v[q, j, 1, pl.ds(i * 16, 16)]
            plsc.addupdate_scatter(deg_loc, [idx16], ones16)
        rem = CHUNK % 16
        if rem:
            lanes = lax.broadcasted_iota(jnp.int32, (16,), 0)
            idx16 = idxp_v[q, j, 1, pl.ds(CHUNK - 16, 16)]
            plsc.addupdate_scatter(deg_loc, [idx16], ones16,
                                   mask=lanes >= (16 - rem))

    def gather(q, j, rows, sem):
        pltpu.async_copy(xw_hbm.at[idxp_v.at[q, j, 0]], rows, sem)

    def wait_gather(q, j, rows, sem):
        pltpu.make_async_copy(xw_hbm.at[idxp_v.at[q, j, 0]], rows, sem).wait()

    def scatter(q, j, rows, sem):
        pltpu.async_copy(rows, acc_sh.at[idxp_v.at[q, j, 1]], sem, add=True)
        count_deg(q, j)

    def wait_scatter(q, j, rows, sem):
        pltpu.make_async_copy(rows, acc_sh.at[idxp_v.at[q, j, 1]], sem).wait()

    pltpu.sync_copy(idxc_hbm.at[wid, pl.ds(0, 2)], idxp_v.at[0])
    gather(0, 0, rowsA_v, gsemA)

    def quad_body(i, _):
        g = i * 4
        pltpu.sync_copy(idxc_hbm.at[wid, pl.ds(g + 2, 2)], idxp_v.at[1])
        gather(0, 1, rowsB_v, gsemB)
        wait_gather(0, 0, rowsA_v, gsemA)
        scatter(0, 0, rowsA_v, ssemA)
        wait_gather(0, 1, rowsB_v, gsemB)
        scatter(0, 1, rowsB_v, ssemB)
        wait_scatter(0, 0, rowsA_v, ssemA)
        gather(1, 0, rowsA_v, gsemA)
        wait_scatter(0, 1, rowsB_v, ssemB)
        gather(1, 1, rowsB_v, gsemB)

        @pl.when(g + 4 < NCHUNK)
        def _():
            pltpu.sync_copy(idxc_hbm.at[wid, pl.ds(g + 4, 2)], idxp_v.at[0])
        wait_gather(1, 0, rowsA_v, gsemA)
        scatter(1, 0, rowsA_v, ssemA)
        wait_gather(1, 1, rowsB_v, gsemB)
        scatter(1, 1, rowsB_v, ssemB)
        wait_scatter(1, 0, rowsA_v, ssemA)

        @pl.when(g + 4 < NCHUNK)
        def _():
            gather(0, 0, rowsA_v, gsemA)
        wait_scatter(1, 1, rowsB_v, ssemB)
        return 0

    lax.fori_loop(0, NCHUNK // 4, quad_body, 0)

    pltpu.sync_copy(deg_loc, deg_hbm.at[c, s])
    plsc.subcore_barrier()

    def out_body(i, _):
        sl = pl.ds(row0 + i * ROW_BLK, ROW_BLK)
        pltpu.sync_copy(acc_sh.at[sl], acc_hbm.at[c, sl])
        return 0
    lax.fori_loop(0, ROWS_PER_TILE // ROW_BLK, out_body, 0)

    @pl.when(s == NS - 1)
    def _():
        sl = pl.ds(NS * ROWS_PER_TILE, ROW_REM)
        pltpu.sync_copy(acc_sh.at[sl], acc_hbm.at[c, sl])


@functools.partial(
    pl.kernel,
    out_type=(
        jax.ShapeDtypeStruct((NC, N, D), jnp.float32),
        jax.ShapeDtypeStruct((NC, NS, N), jnp.float32),
    ),
    mesh=plsc.VectorSubcoreMesh(core_axis_name="c", subcore_axis_name="s",
                                num_cores=NC, num_subcores=NS),
    compiler_params=pltpu.CompilerParams(needs_layout_passes=False),
    scratch_types=[
        pltpu.VMEM((2, 2, 2, CHUNK), jnp.int32),
        pltpu.VMEM((CHUNK, D), jnp.float32),
        pltpu.VMEM((CHUNK, D), jnp.float32),
        pltpu.VMEM((ROW_BLK, D), jnp.float32),
        pltpu.VMEM((N,), jnp.float32),
        pltpu.VMEM_SHARED((N, D), jnp.float32),
        pltpu.SemaphoreType.DMA,
        pltpu.SemaphoreType.DMA,
        pltpu.SemaphoreType.DMA,
        pltpu.SemaphoreType.DMA,
    ],
)
def _sc_edges(xw_hbm, idxc_hbm, acc_hbm, deg_hbm,
              idxp_v, rowsA_v, rowsB_v, zrow_v, deg_loc, acc_sh,
              gsemA, gsemB, ssemA, ssemB):
    _sc_edges_body(xw_hbm, idxc_hbm, acc_hbm, deg_hbm,
                   idxp_v, rowsA_v, rowsB_v, zrow_v, deg_loc, acc_sh,
                   gsemA, gsemB, ssemA, ssemB)


def _tc_finish_body(acc_ref, deg_ref, bias_ref, out_ref):
    acc = acc_ref[0] + acc_ref[1]
    deg = jnp.sum(deg_ref[...], axis=0)[:, None]
    norm = 1.0 / jnp.clip(deg, 1.0, None)
    out_ref[...] = jnp.maximum(acc * norm + bias_ref[...], 0.0)


def _tc_finish(acc, deg, bias):
    return pl.pallas_call(
        _tc_finish_body,
        out_shape=jax.ShapeDtypeStruct((N, D), jnp.float32),
    )(acc, deg.reshape(NC * NS, N), bias.reshape(1, D))


def kernel(X, edge_index, edge_type, bases, comps, bias):
    src = edge_index[0]
    dst = edge_index[1]
    xw, idxc = _tc_prep(X, bases, comps, src, edge_type, dst)
    acc, deg = _sc_edges(xw, idxc.reshape(NW, NCHUNK, 2, CHUNK))
    return _tc_finish(acc, deg, bias)
